# baseline (speedup 1.0000x reference)
import functools

import jax
import jax.numpy as jnp
from jax import lax
from jax.experimental import pallas as pl
from jax.experimental.pallas import tpu as pltpu
from jax.experimental.pallas import tpu_sc as plsc

N = 10000
E = 160000
D = 128
NC = 2
NS = 16
NW = NC * NS

ROWS_PER_TILE = N // NS

AB = 80
ABATCH = (E // NS) // AB

DEG_E_PER_TILE = E // NW
DB = 100
DSUP = 2500
DNSUP = DEG_E_PER_TILE // DSUP
DSCAT = DSUP // DB

AB_E_PER_TILE = E // NW
AB_PAD = 5008

_mesh = plsc.VectorSubcoreMesh(core_axis_name="c", subcore_axis_name="s")
_sc_params = pltpu.CompilerParams(use_tc_tiling_on_sc=False)
_sc_params_nl = pltpu.CompilerParams(use_tc_tiling_on_sc=False, needs_layout_passes=False)


def _wid():
    return lax.axis_index("c") * NS + lax.axis_index("s")


def _deg_body(si_hbm, ri_hbm, rows_s_hbm, rows_r_hbm, z16_hbm, out_hbm,
              acc, idx_s_v, idx_r_v, rs_v, rr_v, sem_i, sem_r, sem_sc):
    cid = lax.axis_index("c")
    tid = lax.axis_index("s")
    wid = cid * NS + tid

    r0 = tid * ROWS_PER_TILE
    pltpu.sync_copy(z16_hbm.at[pl.ds(r0, ROWS_PER_TILE)],
                    acc.at[pl.ds(r0, ROWS_PER_TILE)])
    plsc.subcore_barrier()

    base = wid * DEG_E_PER_TILE

    @pl.loop(0, DNSUP)
    def _sup(sup):
        e0 = base + sup * DSUP
        row0 = e0 // DB
        c1 = pltpu.async_copy(si_hbm.at[pl.ds(row0, DSCAT)], idx_s_v, sem_i)
        c2 = pltpu.async_copy(ri_hbm.at[pl.ds(row0, DSCAT)], idx_r_v, sem_i)
        c3 = pltpu.async_copy(rows_s_hbm.at[pl.ds(e0, DSUP)], rs_v, sem_r)
        c4 = pltpu.async_copy(rows_r_hbm.at[pl.ds(e0, DSUP)], rr_v, sem_r)
        c1.wait(); c2.wait(); c3.wait(); c4.wait()

        @pl.loop(0, DSCAT)
        def _sc(j):
            pltpu.async_copy(rs_v.at[pl.ds(j * DB, DB)],
                             acc.at[idx_s_v.at[j]], sem_sc, add=True)
            pltpu.async_copy(rr_v.at[pl.ds(j * DB, DB)],
                             acc.at[idx_r_v.at[j]], sem_sc, add=True)

        @pl.loop(0, DSCAT)
        def _dr(j):
            pltpu.make_async_copy(rs_v.at[pl.ds(j * DB, DB)],
                                  acc.at[idx_s_v.at[j]], sem_sc).wait()
            pltpu.make_async_copy(rr_v.at[pl.ds(j * DB, DB)],
                                  acc.at[idx_r_v.at[j]], sem_sc).wait()

    plsc.subcore_barrier()
    pltpu.sync_copy(acc.at[pl.ds(r0, ROWS_PER_TILE)],
                    out_hbm.at[cid, pl.ds(r0, ROWS_PER_TILE)])


def _deg_kernel(si2d, ri2d, rows_s, rows_r, z16):
    return pl.kernel(
        _deg_body,
        out_type=jax.ShapeDtypeStruct((NC, N, 16), jnp.float32),
        mesh=_mesh,
        compiler_params=_sc_params,
        scratch_types=[
            pltpu.VMEM_SHARED((N, 16), jnp.float32),
            pltpu.VMEM((DSCAT, DB), jnp.int32),
            pltpu.VMEM((DSCAT, DB), jnp.int32),
            pltpu.VMEM((DSUP, 16), jnp.float32),
            pltpu.VMEM((DSUP, 16), jnp.float32),
            pltpu.SemaphoreType.DMA,
            pltpu.SemaphoreType.DMA,
            pltpu.SemaphoreType.DMA,
        ],
    )(si2d, ri2d, rows_s, rows_r, z16)


def _ab_body(si_hbm, ri_hbm, c2f_hbm, s2f_hbm, c2b_hbm, s2b_hbm,
             dis_s_hbm, dis_r_hbm,
             af_hbm, bf_hbm, ab_hbm, bb_hbm,
             ds_v, dr_v, si_v, ri_v, tf_v, tg_v, th_v, ti_v,
             af_v, bf_v, ab_v, bb_v, sem):
    wid = _wid()
    base = wid * AB_E_PER_TILE

    pltpu.async_copy(dis_s_hbm, ds_v, sem).wait()
    pltpu.async_copy(dis_r_hbm, dr_v, sem).wait()
    c1 = pltpu.async_copy(si_hbm.at[pl.ds(base, AB_PAD)], si_v, sem)
    c2 = pltpu.async_copy(ri_hbm.at[pl.ds(base, AB_PAD)], ri_v, sem)
    c3 = pltpu.async_copy(c2f_hbm.at[pl.ds(base, AB_PAD)], tf_v, sem)
    c4 = pltpu.async_copy(s2f_hbm.at[pl.ds(base, AB_PAD)], tg_v, sem)
    c5 = pltpu.async_copy(c2b_hbm.at[pl.ds(base, AB_PAD)], th_v, sem)
    c6 = pltpu.async_copy(s2b_hbm.at[pl.ds(base, AB_PAD)], ti_v, sem)
    c1.wait(); c2.wait(); c3.wait(); c4.wait(); c5.wait(); c6.wait()

    @pl.loop(0, AB_PAD // 16)
    def _ck(c):
        sl = pl.ds(c * 16, 16)
        sv = si_v[sl]
        rv = ri_v[sl]
        dss = plsc.load_gather(ds_v, [sv])
        dsr = plsc.load_gather(ds_v, [rv])
        drs = plsc.load_gather(dr_v, [sv])
        drr = plsc.load_gather(dr_v, [rv])
        af_v[sl] = dss * tf_v[sl] * drr
        bf_v[sl] = drs * tg_v[sl] * dsr
        ab_v[sl] = dsr * th_v[sl] * drs
        bb_v[sl] = drr * ti_v[sl] * dss

    o1 = pltpu.async_copy(af_v.at[pl.ds(0, AB_E_PER_TILE)],
                          af_hbm.at[pl.ds(base, AB_E_PER_TILE)], sem)
    o2 = pltpu.async_copy(bf_v.at[pl.ds(0, AB_E_PER_TILE)],
                          bf_hbm.at[pl.ds(base, AB_E_PER_TILE)], sem)
    o3 = pltpu.async_copy(ab_v.at[pl.ds(0, AB_E_PER_TILE)],
                          ab_hbm.at[pl.ds(base, AB_E_PER_TILE)], sem)
    o4 = pltpu.async_copy(bb_v.at[pl.ds(0, AB_E_PER_TILE)],
                          bb_hbm.at[pl.ds(base, AB_E_PER_TILE)], sem)
    o1.wait(); o2.wait(); o3.wait(); o4.wait()


def _ab_kernel(si_pad, ri_pad, c2f, s2f, c2b, s2b, dis_s, dis_r):
    ot = jax.ShapeDtypeStruct((E,), jnp.float32)
    return pl.kernel(
        _ab_body,
        out_type=(ot, ot, ot, ot),
        mesh=_mesh,
        compiler_params=_sc_params_nl,
        scratch_types=(
            [pltpu.VMEM((N,), jnp.float32)] * 2
            + [pltpu.VMEM((AB_PAD,), jnp.int32)] * 2
            + [pltpu.VMEM((AB_PAD,), jnp.float32)] * 4
            + [pltpu.VMEM((AB_PAD,), jnp.float32)] * 4
            + [pltpu.SemaphoreType.DMA]
        ),
    )(si_pad, ri_pad, c2f, s2f, c2b, s2b, dis_s, dis_r)


E_PAD = 163840
AB = 64
ABT = E_PAD // NS // AB
HB = ABT // 2


def _scale_rows(xs, ps, wb):
    @plsc.parallel_loop(0, AB, step=1, unroll=2)
    def _row(i):
        wrow = wb[i, pl.ds(0, 16)]
        for c in range(D // 16):
            sl = (i, pl.ds(c * 16, 16))
            ps[sl] = xs[sl] * wrow


def _agg_phase(x_hbm, acc, gi_hbm, sx_hbm, wb_hbm, row0,
               gi_h, sx_h, wb, xs, ps, sg, ss, sw):
    for h in range(2):
        r = row0 + h * HB
        pltpu.sync_copy(gi_hbm.at[pl.ds(r, HB)], gi_h)
        pltpu.sync_copy(sx_hbm.at[pl.ds(r, HB)], sx_h)

        pltpu.async_copy(x_hbm.at[gi_h.at[0]], xs[0], sg[0])
        pltpu.async_copy(x_hbm.at[gi_h.at[1]], xs[1], sg[1])
        pltpu.async_copy(wb_hbm.at[r], wb[0], sw[0])
        pltpu.async_copy(wb_hbm.at[r + 1], wb[1], sw[1])

        def step(k, a):
            pltpu.make_async_copy(x_hbm.at[gi_h.at[k]], xs[a], sg[a]).wait()
            pltpu.make_async_copy(wb_hbm.at[r + k], wb[a], sw[a]).wait()

            def _free():
                pltpu.make_async_copy(ps[a], acc.at[sx_h.at[k]],
                                      ss[a]).wait()

            _maybe(k >= 2, _free)
            _scale_rows(xs[a], ps[a], wb[a])

            def _gnext():
                pltpu.async_copy(x_hbm.at[gi_h.at[k + 2]], xs[a], sg[a])
                pltpu.async_copy(wb_hbm.at[r + k + 2], wb[a], sw[a])

            _maybe(k + 2 <= HB - 1, _gnext)
            pltpu.async_copy(ps[a], acc.at[sx_h.at[k]], ss[a], add=True)

        @pl.loop(0, HB // 2)
        def _it(t):
            step(2 * t, 0)
            step(2 * t + 1, 1)

        pltpu.make_async_copy(ps[0], acc.at[sx_h.at[0]], ss[0]).wait()
        pltpu.make_async_copy(ps[1], acc.at[sx_h.at[1]], ss[1]).wait()


def _maybe(cond, fn):
    if isinstance(cond, bool):
        if cond:
            fn()
    else:
        pl.when(cond)(fn)


def _agg_body(x_hbm, si_hbm, ri_hbm, af_hbm, bf_hbm, ab_hbm, bb_hbm, z_hbm,
              out_hbm,
              acc, gi_h, sx_h, wb0, wb1, xs0, xs1, ps0, ps1,
              sg0, sg1, ss0, ss1, sw0, sw1):
    cid = lax.axis_index("c")
    tid = lax.axis_index("s")

    r0 = tid * ROWS_PER_TILE
    pltpu.sync_copy(z_hbm.at[pl.ds(r0, ROWS_PER_TILE)],
                    acc.at[pl.ds(r0, ROWS_PER_TILE)])
    plsc.subcore_barrier()

    row0 = tid * ABT
    xs = (xs0, xs1)
    ps = (ps0, ps1)
    wb = (wb0, wb1)
    sg = (sg0, sg1)
    ss = (ss0, ss1)
    sw = (sw0, sw1)

    @pl.when(cid == 0)
    def _sc0():
        _agg_phase(x_hbm, acc, si_hbm, ri_hbm, af_hbm, row0,
                   gi_h, sx_h, wb, xs, ps, sg, ss, sw)
        _agg_phase(x_hbm, acc, ri_hbm, si_hbm, ab_hbm, row0,
                   gi_h, sx_h, wb, xs, ps, sg, ss, sw)

    @pl.when(cid == 1)
    def _sc1():
        _agg_phase(x_hbm, acc, si_hbm, ri_hbm, bf_hbm, row0,
                   gi_h, sx_h, wb, xs, ps, sg, ss, sw)
        _agg_phase(x_hbm, acc, ri_hbm, si_hbm, bb_hbm, row0,
                   gi_h, sx_h, wb, xs, ps, sg, ss, sw)

    plsc.subcore_barrier()
    pltpu.sync_copy(acc.at[pl.ds(r0, ROWS_PER_TILE)],
                    out_hbm.at[cid, pl.ds(r0, ROWS_PER_TILE)])


def _agg_kernel(x, si2d, ri2d, af2d, bf2d, ab2d, bb2d, z128):
    return pl.kernel(
        _agg_body,
        out_type=jax.ShapeDtypeStruct((NC, N, D), jnp.float32),
        mesh=_mesh,
        compiler_params=_sc_params,
        scratch_types=(
            [pltpu.VMEM_SHARED((N, D), jnp.float32)]
            + [pltpu.VMEM((HB, AB), jnp.int32)] * 2
            + [pltpu.VMEM((AB, 16), jnp.float32)] * 2
            + [pltpu.VMEM((AB, D), jnp.float32)] * 4
            + [pltpu.SemaphoreType.DMA] * 6
        ),
    )(x, si2d, ri2d, af2d, bf2d, ab2d, bb2d, z128)


BN = 1000


def _dense1_body(s2d_ref, d2s_ref, x_ref, dd_ref, ws_ref, wd_ref,
                 bs_ref, bd_ref, o_ref):
    y = x_ref[...] * dd_ref[...]
    a = s2d_ref[...] + y
    b = d2s_ref[...] + y
    h = 0.5 * (lax.dot_general(a, ws_ref[...], (((1,), (0,)), ((), ())),
                               precision=lax.Precision.HIGHEST,
                               preferred_element_type=jnp.float32)
               + bs_ref[...])
    h = h + 0.5 * (lax.dot_general(b, wd_ref[...], (((1,), (0,)), ((), ())),
                                   precision=lax.Precision.HIGHEST,
                                   preferred_element_type=jnp.float32)
                   + bd_ref[...])
    o_ref[...] = jnp.maximum(h, 0.0)


def _dense2_body(s2d_ref, d2s_ref, x_ref, dd_ref, ws_ref, wd_ref,
                 bs_ref, bd_ref, wr_ref, br_ref, o_ref):
    y = x_ref[...] * dd_ref[...]
    a = s2d_ref[...] + y
    b = d2s_ref[...] + y
    h = 0.5 * (lax.dot_general(a, ws_ref[...], (((1,), (0,)), ((), ())),
                               precision=lax.Precision.HIGHEST,
                               preferred_element_type=jnp.float32)
               + bs_ref[...])
    h = h + 0.5 * (lax.dot_general(b, wd_ref[...], (((1,), (0,)), ((), ())),
                                   precision=lax.Precision.HIGHEST,
                                   preferred_element_type=jnp.float32)
                   + bd_ref[...])
    h = jnp.maximum(h, 0.0)
    o_ref[...] = lax.dot_general(h, wr_ref[...], (((1,), (0,)), ((), ())),
                                 precision=lax.Precision.HIGHEST,
                                 preferred_element_type=jnp.float32) + br_ref[...]


def _row_spec():
    return pl.BlockSpec((BN, D), lambda i: (i, 0))


def _w_spec():
    return pl.BlockSpec((D, D), lambda i: (0, 0))


def _b_spec():
    return pl.BlockSpec((1, D), lambda i: (0, 0))


def _dense1(s2d, d2s, x, dd2, ws, wd, bs, bd):
    return pl.pallas_call(
        _dense1_body,
        grid=(N // BN,),
        in_specs=[_row_spec(), _row_spec(), _row_spec(),
                  pl.BlockSpec((BN, 1), lambda i: (i, 0)),
                  _w_spec(), _w_spec(), _b_spec(), _b_spec()],
        out_specs=_row_spec(),
        out_shape=jax.ShapeDtypeStruct((N, D), jnp.float32),
    )(s2d, d2s, x, dd2, ws, wd, bs.reshape(1, D), bd.reshape(1, D))


def _dense2(s2d, d2s, x, dd2, ws, wd, bs, bd, wr, br):
    return pl.pallas_call(
        _dense2_body,
        grid=(N // BN,),
        in_specs=[_row_spec(), _row_spec(), _row_spec(),
                  pl.BlockSpec((BN, 1), lambda i: (i, 0)),
                  _w_spec(), _w_spec(), _b_spec(), _b_spec(),
                  _w_spec(), _b_spec()],
        out_specs=_row_spec(),
        out_shape=jax.ShapeDtypeStruct((N, D), jnp.float32),
    )(s2d, d2s, x, dd2, ws, wd, bs.reshape(1, D), bd.reshape(1, D),
      wr, br.reshape(1, D))


def kernel(x, edge_index, theta, W_s2d_0, W_d2s_0, b_s2d_0, b_d2s_0,
           W_s2d_1, W_d2s_1, b_s2d_1, b_d2s_1, W_read, b_read):
    si = edge_index[0].astype(jnp.int32)
    ri = edge_index[1].astype(jnp.int32)

    c2f = jnp.cos(theta) ** 2
    s2f = jnp.sin(theta) ** 2
    c2b = jnp.cos(jnp.pi / 2 - theta) ** 2
    s2b = jnp.sin(jnp.pi / 2 - theta) ** 2

    zpad = jnp.zeros((E, 14), jnp.float32)
    rows_s = jnp.concatenate([c2f[:, None], s2f[:, None], zpad], axis=1)
    rows_r = jnp.concatenate([c2b[:, None], s2b[:, None], zpad], axis=1)

    si2d = si.reshape(E // DB, DB)
    ri2d = ri.reshape(E // DB, DB)
    z16 = jnp.zeros((N, 16), jnp.float32)

    degp = _deg_kernel(si2d, ri2d, rows_s, rows_r, z16)
    degs = degp[0] + degp[1]
    deg_s = degs[:, 0] + 1.0 + 1e-12
    deg_r = degs[:, 1] + 1.0 + 1e-12
    dis_s = jnp.where(deg_s > 1e11, 0.0, lax.rsqrt(deg_s))
    dis_r = jnp.where(deg_r > 1e11, 0.0, lax.rsqrt(deg_r))
    dd = dis_s * dis_r

    ipad = jnp.zeros((16,), jnp.int32)
    fpad = jnp.zeros((16,), jnp.float32)
    si_pad = jnp.concatenate([si, ipad])
    ri_pad = jnp.concatenate([ri, ipad])
    c2f_p = jnp.concatenate([c2f, fpad])
    s2f_p = jnp.concatenate([s2f, fpad])
    c2b_p = jnp.concatenate([c2b, fpad])
    s2b_p = jnp.concatenate([s2b, fpad])

    af, bf, ab_, bb = _ab_kernel(si_pad, ri_pad, c2f_p, s2f_p, c2b_p, s2b_p,
                                 dis_s, dis_r)

    z128 = jnp.zeros((N, D), jnp.float32)
    dd2 = dd.reshape(N, 1)

    epad_i = jnp.zeros((E_PAD - E,), jnp.int32)
    epad_f = jnp.zeros((E_PAD - E,), jnp.float32)
    si2a = jnp.concatenate([si, epad_i]).reshape(E_PAD // AB, AB)
    ri2a = jnp.concatenate([ri, epad_i]).reshape(E_PAD // AB, AB)
    def _wb(w):
        w = jnp.concatenate([w, epad_f])
        return jnp.broadcast_to(w[:, None], (E_PAD, 16)).reshape(
            E_PAD // AB, AB, 16)

    af2 = _wb(af)
    bf2 = _wb(bf)
    ab2 = _wb(ab_)
    bb2 = _wb(bb)

    aggs = _agg_kernel(x, si2a, ri2a, af2, bf2, ab2, bb2, z128)
    x1 = _dense1(aggs[0], aggs[1], x, dd2, W_s2d_0, W_d2s_0, b_s2d_0, b_d2s_0)

    aggs2 = _agg_kernel(x1, si2a, ri2a, af2, bf2, ab2, bb2, z128)
    out = _dense2(aggs2[0], aggs2[1], x1, dd2, W_s2d_1, W_d2s_1,
                  b_s2d_1, b_d2s_1, W_read, b_read)
    return out


# speedup vs baseline: 2.5018x; 2.5018x over previous
import functools

import jax
import jax.numpy as jnp
from jax import lax
from jax.experimental import pallas as pl
from jax.experimental.pallas import tpu as pltpu
from jax.experimental.pallas import tpu_sc as plsc

N = 10000
E = 160000
D = 128
NC = 2
NS = 16
NW = NC * NS

ROWS_PER_TILE = N // NS

AB = 80
ABATCH = (E // NS) // AB

DEG_E_PER_TILE = E // NW
DB = 100
DSUP = 2500
DNSUP = DEG_E_PER_TILE // DSUP
DSCAT = DSUP // DB

AB_E_PER_TILE = E // NW
AB_PAD = 5008

_mesh = plsc.VectorSubcoreMesh(core_axis_name="c", subcore_axis_name="s")
_sc_params = pltpu.CompilerParams(use_tc_tiling_on_sc=False)
_sc_params_nl = pltpu.CompilerParams(use_tc_tiling_on_sc=False, needs_layout_passes=False)


def _wid():
    return lax.axis_index("c") * NS + lax.axis_index("s")


def _deg_body(si_hbm, ri_hbm, rows_s_hbm, rows_r_hbm, z16_hbm, out_hbm,
              acc, idx_s_v, idx_r_v, rs_v, rr_v, sem_i, sem_r, sem_sc):
    cid = lax.axis_index("c")
    tid = lax.axis_index("s")
    wid = cid * NS + tid

    r0 = tid * ROWS_PER_TILE
    pltpu.sync_copy(z16_hbm.at[pl.ds(r0, ROWS_PER_TILE)],
                    acc.at[pl.ds(r0, ROWS_PER_TILE)])
    plsc.subcore_barrier()

    base = wid * DEG_E_PER_TILE

    @pl.loop(0, DNSUP)
    def _sup(sup):
        e0 = base + sup * DSUP
        row0 = e0 // DB
        c1 = pltpu.async_copy(si_hbm.at[pl.ds(row0, DSCAT)], idx_s_v, sem_i)
        c2 = pltpu.async_copy(ri_hbm.at[pl.ds(row0, DSCAT)], idx_r_v, sem_i)
        c3 = pltpu.async_copy(rows_s_hbm.at[pl.ds(e0, DSUP)], rs_v, sem_r)
        c4 = pltpu.async_copy(rows_r_hbm.at[pl.ds(e0, DSUP)], rr_v, sem_r)
        c1.wait(); c2.wait(); c3.wait(); c4.wait()

        @pl.loop(0, DSCAT)
        def _sc(j):
            pltpu.async_copy(rs_v.at[pl.ds(j * DB, DB)],
                             acc.at[idx_s_v.at[j]], sem_sc, add=True)
            pltpu.async_copy(rr_v.at[pl.ds(j * DB, DB)],
                             acc.at[idx_r_v.at[j]], sem_sc, add=True)

        @pl.loop(0, DSCAT)
        def _dr(j):
            pltpu.make_async_copy(rs_v.at[pl.ds(j * DB, DB)],
                                  acc.at[idx_s_v.at[j]], sem_sc).wait()
            pltpu.make_async_copy(rr_v.at[pl.ds(j * DB, DB)],
                                  acc.at[idx_r_v.at[j]], sem_sc).wait()

    plsc.subcore_barrier()
    pltpu.sync_copy(acc.at[pl.ds(r0, ROWS_PER_TILE)],
                    out_hbm.at[cid, pl.ds(r0, ROWS_PER_TILE)])


def _deg_kernel(si2d, ri2d, rows_s, rows_r, z16):
    return pl.kernel(
        _deg_body,
        out_type=jax.ShapeDtypeStruct((NC, N, 16), jnp.float32),
        mesh=_mesh,
        compiler_params=_sc_params,
        scratch_types=[
            pltpu.VMEM_SHARED((N, 16), jnp.float32),
            pltpu.VMEM((DSCAT, DB), jnp.int32),
            pltpu.VMEM((DSCAT, DB), jnp.int32),
            pltpu.VMEM((DSUP, 16), jnp.float32),
            pltpu.VMEM((DSUP, 16), jnp.float32),
            pltpu.SemaphoreType.DMA,
            pltpu.SemaphoreType.DMA,
            pltpu.SemaphoreType.DMA,
        ],
    )(si2d, ri2d, rows_s, rows_r, z16)


def _ab_body(si_hbm, ri_hbm, c2f_hbm, s2f_hbm, c2b_hbm, s2b_hbm,
             dis_s_hbm, dis_r_hbm,
             af_hbm, bf_hbm, ab_hbm, bb_hbm,
             ds_v, dr_v, si_v, ri_v, tf_v, tg_v, th_v, ti_v,
             af_v, bf_v, ab_v, bb_v, sem):
    wid = _wid()
    base = wid * AB_E_PER_TILE

    pltpu.async_copy(dis_s_hbm, ds_v, sem).wait()
    pltpu.async_copy(dis_r_hbm, dr_v, sem).wait()
    c1 = pltpu.async_copy(si_hbm.at[pl.ds(base, AB_PAD)], si_v, sem)
    c2 = pltpu.async_copy(ri_hbm.at[pl.ds(base, AB_PAD)], ri_v, sem)
    c3 = pltpu.async_copy(c2f_hbm.at[pl.ds(base, AB_PAD)], tf_v, sem)
    c4 = pltpu.async_copy(s2f_hbm.at[pl.ds(base, AB_PAD)], tg_v, sem)
    c5 = pltpu.async_copy(c2b_hbm.at[pl.ds(base, AB_PAD)], th_v, sem)
    c6 = pltpu.async_copy(s2b_hbm.at[pl.ds(base, AB_PAD)], ti_v, sem)
    c1.wait(); c2.wait(); c3.wait(); c4.wait(); c5.wait(); c6.wait()

    @pl.loop(0, AB_PAD // 16)
    def _ck(c):
        sl = pl.ds(c * 16, 16)
        sv = si_v[sl]
        rv = ri_v[sl]
        dss = plsc.load_gather(ds_v, [sv])
        dsr = plsc.load_gather(ds_v, [rv])
        drs = plsc.load_gather(dr_v, [sv])
        drr = plsc.load_gather(dr_v, [rv])
        af_v[sl] = dss * tf_v[sl] * drr
        bf_v[sl] = drs * tg_v[sl] * dsr
        ab_v[sl] = dsr * th_v[sl] * drs
        bb_v[sl] = drr * ti_v[sl] * dss

    o1 = pltpu.async_copy(af_v.at[pl.ds(0, AB_E_PER_TILE)],
                          af_hbm.at[pl.ds(base, AB_E_PER_TILE)], sem)
    o2 = pltpu.async_copy(bf_v.at[pl.ds(0, AB_E_PER_TILE)],
                          bf_hbm.at[pl.ds(base, AB_E_PER_TILE)], sem)
    o3 = pltpu.async_copy(ab_v.at[pl.ds(0, AB_E_PER_TILE)],
                          ab_hbm.at[pl.ds(base, AB_E_PER_TILE)], sem)
    o4 = pltpu.async_copy(bb_v.at[pl.ds(0, AB_E_PER_TILE)],
                          bb_hbm.at[pl.ds(base, AB_E_PER_TILE)], sem)
    o1.wait(); o2.wait(); o3.wait(); o4.wait()


def _ab_kernel(si_pad, ri_pad, c2f, s2f, c2b, s2b, dis_s, dis_r):
    ot = jax.ShapeDtypeStruct((E,), jnp.float32)
    return pl.kernel(
        _ab_body,
        out_type=(ot, ot, ot, ot),
        mesh=_mesh,
        compiler_params=_sc_params_nl,
        scratch_types=(
            [pltpu.VMEM((N,), jnp.float32)] * 2
            + [pltpu.VMEM((AB_PAD,), jnp.int32)] * 2
            + [pltpu.VMEM((AB_PAD,), jnp.float32)] * 4
            + [pltpu.VMEM((AB_PAD,), jnp.float32)] * 4
            + [pltpu.SemaphoreType.DMA]
        ),
    )(si_pad, ri_pad, c2f, s2f, c2b, s2b, dis_s, dis_r)


AB = 80
ABATCH = (E // NS) // AB


def _scale_rows(xs, w_v, k):
    @pl.loop(0, AB, step=16)
    def _grp(g):
        wv = w_v[k, pl.ds(g, 16)]
        for j in range(16):
            wi = wv[j]
            for c in range(D // 16):
                sl = (g + j, pl.ds(c * 16, 16))
                xs[sl] = xs[sl] * wi


def _agg_phase(x_hbm, acc, g_idx, s_idx, w_v, xs0, xs1,
               sem_g0, sem_g1, sem_s0, sem_s1):
    NPAIR = ABATCH // 2
    pltpu.async_copy(x_hbm.at[g_idx.at[0]], xs0, sem_g0)
    pltpu.async_copy(x_hbm.at[g_idx.at[1]], xs1, sem_g1)

    @pl.loop(0, NPAIR)
    def _it(t):
        k0 = 2 * t
        k1 = 2 * t + 1
        pltpu.make_async_copy(x_hbm.at[g_idx.at[k0]], xs0, sem_g0).wait()
        _scale_rows(xs0, w_v, k0)
        pltpu.async_copy(xs0, acc.at[s_idx.at[k0]], sem_s0, add=True)

        pltpu.make_async_copy(x_hbm.at[g_idx.at[k1]], xs1, sem_g1).wait()
        _scale_rows(xs1, w_v, k1)
        pltpu.async_copy(xs1, acc.at[s_idx.at[k1]], sem_s1, add=True)

        pltpu.make_async_copy(xs0, acc.at[s_idx.at[k0]], sem_s0).wait()
        pltpu.async_copy(x_hbm.at[g_idx.at[k0 + 2]], xs0, sem_g0)

        @pl.when(t < NPAIR - 1)
        def _next():
            pltpu.make_async_copy(xs1, acc.at[s_idx.at[k1]], sem_s1).wait()
            pltpu.async_copy(x_hbm.at[g_idx.at[k1 + 2]], xs1, sem_g1)

    k = ABATCH - 1
    pltpu.make_async_copy(x_hbm.at[g_idx.at[k]], xs0, sem_g0).wait()
    _scale_rows(xs0, w_v, k)
    pltpu.async_copy(xs0, acc.at[s_idx.at[k]], sem_s0, add=True)
    pltpu.make_async_copy(xs0, acc.at[s_idx.at[k]], sem_s0).wait()
    pltpu.make_async_copy(xs1, acc.at[s_idx.at[ABATCH - 2]], sem_s1).wait()


def _agg_body(x_hbm, si_hbm, ri_hbm, af_hbm, bf_hbm, ab_hbm, bb_hbm, z_hbm,
              out_hbm,
              acc, si_v, ri_v, w_v, xs0, xs1,
              sem_i, sem_g0, sem_g1, sem_s0, sem_s1):
    cid = lax.axis_index("c")
    tid = lax.axis_index("s")

    r0 = tid * ROWS_PER_TILE
    pltpu.sync_copy(z_hbm.at[pl.ds(r0, ROWS_PER_TILE)],
                    acc.at[pl.ds(r0, ROWS_PER_TILE)])
    plsc.subcore_barrier()

    row0 = tid * ABATCH
    c1 = pltpu.async_copy(si_hbm.at[pl.ds(row0, ABATCH)], si_v, sem_i)
    c2 = pltpu.async_copy(ri_hbm.at[pl.ds(row0, ABATCH)], ri_v, sem_i)
    c1.wait(); c2.wait()

    @pl.when(cid == 0)
    def _sc0():
        pltpu.async_copy(af_hbm.at[pl.ds(row0, ABATCH)], w_v, sem_i).wait()
        _agg_phase(x_hbm, acc, si_v, ri_v, w_v, xs0, xs1,
                   sem_g0, sem_g1, sem_s0, sem_s1)
        pltpu.async_copy(ab_hbm.at[pl.ds(row0, ABATCH)], w_v, sem_i).wait()
        _agg_phase(x_hbm, acc, ri_v, si_v, w_v, xs0, xs1,
                   sem_g0, sem_g1, sem_s0, sem_s1)

    @pl.when(cid == 1)
    def _sc1():
        pltpu.async_copy(bf_hbm.at[pl.ds(row0, ABATCH)], w_v, sem_i).wait()
        _agg_phase(x_hbm, acc, si_v, ri_v, w_v, xs0, xs1,
                   sem_g0, sem_g1, sem_s0, sem_s1)
        pltpu.async_copy(bb_hbm.at[pl.ds(row0, ABATCH)], w_v, sem_i).wait()
        _agg_phase(x_hbm, acc, ri_v, si_v, w_v, xs0, xs1,
                   sem_g0, sem_g1, sem_s0, sem_s1)

    plsc.subcore_barrier()
    pltpu.sync_copy(acc.at[pl.ds(r0, ROWS_PER_TILE)],
                    out_hbm.at[cid, pl.ds(r0, ROWS_PER_TILE)])


def _agg_kernel(x, si2d, ri2d, af2d, bf2d, ab2d, bb2d, z128):
    return pl.kernel(
        _agg_body,
        out_type=jax.ShapeDtypeStruct((NC, N, D), jnp.float32),
        mesh=_mesh,
        compiler_params=_sc_params,
        scratch_types=[
            pltpu.VMEM_SHARED((N, D), jnp.float32),
            pltpu.VMEM((ABATCH, AB), jnp.int32),
            pltpu.VMEM((ABATCH, AB), jnp.int32),
            pltpu.VMEM((ABATCH, AB), jnp.float32),
            pltpu.VMEM((AB, D), jnp.float32),
            pltpu.VMEM((AB, D), jnp.float32),
            pltpu.SemaphoreType.DMA,
            pltpu.SemaphoreType.DMA,
            pltpu.SemaphoreType.DMA,
            pltpu.SemaphoreType.DMA,
            pltpu.SemaphoreType.DMA,
        ],
    )(x, si2d, ri2d, af2d, bf2d, ab2d, bb2d, z128)


BN = 1000


def _dense1_body(s2d_ref, d2s_ref, x_ref, dd_ref, ws_ref, wd_ref,
                 bs_ref, bd_ref, o_ref):
    y = x_ref[...] * dd_ref[...]
    a = s2d_ref[...] + y
    b = d2s_ref[...] + y
    h = 0.5 * (lax.dot_general(a, ws_ref[...], (((1,), (0,)), ((), ())),
                               precision=lax.Precision.HIGHEST,
                               preferred_element_type=jnp.float32)
               + bs_ref[...])
    h = h + 0.5 * (lax.dot_general(b, wd_ref[...], (((1,), (0,)), ((), ())),
                                   precision=lax.Precision.HIGHEST,
                                   preferred_element_type=jnp.float32)
                   + bd_ref[...])
    o_ref[...] = jnp.maximum(h, 0.0)


def _dense2_body(s2d_ref, d2s_ref, x_ref, dd_ref, ws_ref, wd_ref,
                 bs_ref, bd_ref, wr_ref, br_ref, o_ref):
    y = x_ref[...] * dd_ref[...]
    a = s2d_ref[...] + y
    b = d2s_ref[...] + y
    h = 0.5 * (lax.dot_general(a, ws_ref[...], (((1,), (0,)), ((), ())),
                               precision=lax.Precision.HIGHEST,
                               preferred_element_type=jnp.float32)
               + bs_ref[...])
    h = h + 0.5 * (lax.dot_general(b, wd_ref[...], (((1,), (0,)), ((), ())),
                                   precision=lax.Precision.HIGHEST,
                                   preferred_element_type=jnp.float32)
                   + bd_ref[...])
    h = jnp.maximum(h, 0.0)
    o_ref[...] = lax.dot_general(h, wr_ref[...], (((1,), (0,)), ((), ())),
                                 precision=lax.Precision.HIGHEST,
                                 preferred_element_type=jnp.float32) + br_ref[...]


def _row_spec():
    return pl.BlockSpec((BN, D), lambda i: (i, 0))


def _w_spec():
    return pl.BlockSpec((D, D), lambda i: (0, 0))


def _b_spec():
    return pl.BlockSpec((1, D), lambda i: (0, 0))


def _dense1(s2d, d2s, x, dd2, ws, wd, bs, bd):
    return pl.pallas_call(
        _dense1_body,
        grid=(N // BN,),
        in_specs=[_row_spec(), _row_spec(), _row_spec(),
                  pl.BlockSpec((BN, 1), lambda i: (i, 0)),
                  _w_spec(), _w_spec(), _b_spec(), _b_spec()],
        out_specs=_row_spec(),
        out_shape=jax.ShapeDtypeStruct((N, D), jnp.float32),
    )(s2d, d2s, x, dd2, ws, wd, bs.reshape(1, D), bd.reshape(1, D))


def _dense2(s2d, d2s, x, dd2, ws, wd, bs, bd, wr, br):
    return pl.pallas_call(
        _dense2_body,
        grid=(N // BN,),
        in_specs=[_row_spec(), _row_spec(), _row_spec(),
                  pl.BlockSpec((BN, 1), lambda i: (i, 0)),
                  _w_spec(), _w_spec(), _b_spec(), _b_spec(),
                  _w_spec(), _b_spec()],
        out_specs=_row_spec(),
        out_shape=jax.ShapeDtypeStruct((N, D), jnp.float32),
    )(s2d, d2s, x, dd2, ws, wd, bs.reshape(1, D), bd.reshape(1, D),
      wr, br.reshape(1, D))


def kernel(x, edge_index, theta, W_s2d_0, W_d2s_0, b_s2d_0, b_d2s_0,
           W_s2d_1, W_d2s_1, b_s2d_1, b_d2s_1, W_read, b_read):
    si = edge_index[0].astype(jnp.int32)
    ri = edge_index[1].astype(jnp.int32)

    c2f = jnp.cos(theta) ** 2
    s2f = jnp.sin(theta) ** 2
    c2b = jnp.cos(jnp.pi / 2 - theta) ** 2
    s2b = jnp.sin(jnp.pi / 2 - theta) ** 2

    zpad = jnp.zeros((E, 14), jnp.float32)
    rows_s = jnp.concatenate([c2f[:, None], s2f[:, None], zpad], axis=1)
    rows_r = jnp.concatenate([c2b[:, None], s2b[:, None], zpad], axis=1)

    si2d = si.reshape(E // DB, DB)
    ri2d = ri.reshape(E // DB, DB)
    z16 = jnp.zeros((N, 16), jnp.float32)

    degp = _deg_kernel(si2d, ri2d, rows_s, rows_r, z16)
    degs = degp[0] + degp[1]
    deg_s = degs[:, 0] + 1.0 + 1e-12
    deg_r = degs[:, 1] + 1.0 + 1e-12
    dis_s = jnp.where(deg_s > 1e11, 0.0, lax.rsqrt(deg_s))
    dis_r = jnp.where(deg_r > 1e11, 0.0, lax.rsqrt(deg_r))
    dd = dis_s * dis_r

    ipad = jnp.zeros((16,), jnp.int32)
    fpad = jnp.zeros((16,), jnp.float32)
    si_pad = jnp.concatenate([si, ipad])
    ri_pad = jnp.concatenate([ri, ipad])
    c2f_p = jnp.concatenate([c2f, fpad])
    s2f_p = jnp.concatenate([s2f, fpad])
    c2b_p = jnp.concatenate([c2b, fpad])
    s2b_p = jnp.concatenate([s2b, fpad])

    af, bf, ab_, bb = _ab_kernel(si_pad, ri_pad, c2f_p, s2f_p, c2b_p, s2b_p,
                                 dis_s, dis_r)

    z128 = jnp.zeros((N, D), jnp.float32)
    dd2 = dd.reshape(N, 1)

    si2a = si.reshape(E // AB, AB)
    ri2a = ri.reshape(E // AB, AB)
    af2 = af.reshape(E // AB, AB)
    bf2 = bf.reshape(E // AB, AB)
    ab2 = ab_.reshape(E // AB, AB)
    bb2 = bb.reshape(E // AB, AB)

    aggs = _agg_kernel(x, si2a, ri2a, af2, bf2, ab2, bb2, z128)
    x1 = _dense1(aggs[0], aggs[1], x, dd2, W_s2d_0, W_d2s_0, b_s2d_0, b_d2s_0)

    aggs2 = _agg_kernel(x1, si2a, ri2a, af2, bf2, ab2, bb2, z128)
    out = _dense2(aggs2[0], aggs2[1], x1, dd2, W_s2d_1, W_d2s_1,
                  b_s2d_1, b_d2s_1, W_read, b_read)
    return out


# speedup vs baseline: 2.5118x; 1.0040x over previous
import functools

import jax
import jax.numpy as jnp
from jax import lax
from jax.experimental import pallas as pl
from jax.experimental.pallas import tpu as pltpu
from jax.experimental.pallas import tpu_sc as plsc

N = 10000
E = 160000
D = 128
NC = 2
NS = 16
NW = NC * NS

ROWS_PER_TILE = N // NS

AB = 100
ABATCH = (E // NS) // AB
ABH = ABATCH // 2

DEG_E_PER_TILE = E // NW
DB = 100
DSUP = 2500
DNSUP = DEG_E_PER_TILE // DSUP
DSCAT = DSUP // DB

AB_E_PER_TILE = E // NW
AB_PAD = 5008

_mesh = plsc.VectorSubcoreMesh(core_axis_name="c", subcore_axis_name="s")
_sc_params = pltpu.CompilerParams(use_tc_tiling_on_sc=False)
_sc_params_nl = pltpu.CompilerParams(use_tc_tiling_on_sc=False, needs_layout_passes=False)


def _wid():
    return lax.axis_index("c") * NS + lax.axis_index("s")


def _deg_body(si_hbm, ri_hbm, rows_s_hbm, rows_r_hbm, z16_hbm, out_hbm,
              acc, idx_s_v, idx_r_v, rs_v, rr_v, sem_i, sem_r, sem_sc):
    cid = lax.axis_index("c")
    tid = lax.axis_index("s")
    wid = cid * NS + tid

    r0 = tid * ROWS_PER_TILE
    pltpu.sync_copy(z16_hbm.at[pl.ds(r0, ROWS_PER_TILE)],
                    acc.at[pl.ds(r0, ROWS_PER_TILE)])
    plsc.subcore_barrier()

    base = wid * DEG_E_PER_TILE

    @pl.loop(0, DNSUP)
    def _sup(sup):
        e0 = base + sup * DSUP
        row0 = e0 // DB
        c1 = pltpu.async_copy(si_hbm.at[pl.ds(row0, DSCAT)], idx_s_v, sem_i)
        c2 = pltpu.async_copy(ri_hbm.at[pl.ds(row0, DSCAT)], idx_r_v, sem_i)
        c3 = pltpu.async_copy(rows_s_hbm.at[pl.ds(e0, DSUP)], rs_v, sem_r)
        c4 = pltpu.async_copy(rows_r_hbm.at[pl.ds(e0, DSUP)], rr_v, sem_r)
        c1.wait(); c2.wait(); c3.wait(); c4.wait()

        @pl.loop(0, DSCAT)
        def _sc(j):
            pltpu.async_copy(rs_v.at[pl.ds(j * DB, DB)],
                             acc.at[idx_s_v.at[j]], sem_sc, add=True)
            pltpu.async_copy(rr_v.at[pl.ds(j * DB, DB)],
                             acc.at[idx_r_v.at[j]], sem_sc, add=True)

        @pl.loop(0, DSCAT)
        def _dr(j):
            pltpu.make_async_copy(rs_v.at[pl.ds(j * DB, DB)],
                                  acc.at[idx_s_v.at[j]], sem_sc).wait()
            pltpu.make_async_copy(rr_v.at[pl.ds(j * DB, DB)],
                                  acc.at[idx_r_v.at[j]], sem_sc).wait()

    plsc.subcore_barrier()
    pltpu.sync_copy(acc.at[pl.ds(r0, ROWS_PER_TILE)],
                    out_hbm.at[cid, pl.ds(r0, ROWS_PER_TILE)])


def _deg_kernel(si2d, ri2d, rows_s, rows_r, z16):
    return pl.kernel(
        _deg_body,
        out_type=jax.ShapeDtypeStruct((NC, N, 16), jnp.float32),
        mesh=_mesh,
        compiler_params=_sc_params,
        scratch_types=[
            pltpu.VMEM_SHARED((N, 16), jnp.float32),
            pltpu.VMEM((DSCAT, DB), jnp.int32),
            pltpu.VMEM((DSCAT, DB), jnp.int32),
            pltpu.VMEM((DSUP, 16), jnp.float32),
            pltpu.VMEM((DSUP, 16), jnp.float32),
            pltpu.SemaphoreType.DMA,
            pltpu.SemaphoreType.DMA,
            pltpu.SemaphoreType.DMA,
        ],
    )(si2d, ri2d, rows_s, rows_r, z16)


def _ab_body(si_hbm, ri_hbm, c2f_hbm, s2f_hbm, c2b_hbm, s2b_hbm,
             dis_s_hbm, dis_r_hbm,
             af_hbm, bf_hbm, ab_hbm, bb_hbm,
             ds_v, dr_v, si_v, ri_v, tf_v, tg_v, th_v, ti_v,
             af_v, bf_v, ab_v, bb_v, sem):
    wid = _wid()
    base = wid * AB_E_PER_TILE

    pltpu.async_copy(dis_s_hbm, ds_v, sem).wait()
    pltpu.async_copy(dis_r_hbm, dr_v, sem).wait()
    c1 = pltpu.async_copy(si_hbm.at[pl.ds(base, AB_PAD)], si_v, sem)
    c2 = pltpu.async_copy(ri_hbm.at[pl.ds(base, AB_PAD)], ri_v, sem)
    c3 = pltpu.async_copy(c2f_hbm.at[pl.ds(base, AB_PAD)], tf_v, sem)
    c4 = pltpu.async_copy(s2f_hbm.at[pl.ds(base, AB_PAD)], tg_v, sem)
    c5 = pltpu.async_copy(c2b_hbm.at[pl.ds(base, AB_PAD)], th_v, sem)
    c6 = pltpu.async_copy(s2b_hbm.at[pl.ds(base, AB_PAD)], ti_v, sem)
    c1.wait(); c2.wait(); c3.wait(); c4.wait(); c5.wait(); c6.wait()

    @pl.loop(0, AB_PAD // 16)
    def _ck(c):
        sl = pl.ds(c * 16, 16)
        sv = si_v[sl]
        rv = ri_v[sl]
        dss = plsc.load_gather(ds_v, [sv])
        dsr = plsc.load_gather(ds_v, [rv])
        drs = plsc.load_gather(dr_v, [sv])
        drr = plsc.load_gather(dr_v, [rv])
        af_v[sl] = dss * tf_v[sl] * drr
        bf_v[sl] = drs * tg_v[sl] * dsr
        ab_v[sl] = dsr * th_v[sl] * drs
        bb_v[sl] = drr * ti_v[sl] * dss

    o1 = pltpu.async_copy(af_v.at[pl.ds(0, AB_E_PER_TILE)],
                          af_hbm.at[pl.ds(base, AB_E_PER_TILE)], sem)
    o2 = pltpu.async_copy(bf_v.at[pl.ds(0, AB_E_PER_TILE)],
                          bf_hbm.at[pl.ds(base, AB_E_PER_TILE)], sem)
    o3 = pltpu.async_copy(ab_v.at[pl.ds(0, AB_E_PER_TILE)],
                          ab_hbm.at[pl.ds(base, AB_E_PER_TILE)], sem)
    o4 = pltpu.async_copy(bb_v.at[pl.ds(0, AB_E_PER_TILE)],
                          bb_hbm.at[pl.ds(base, AB_E_PER_TILE)], sem)
    o1.wait(); o2.wait(); o3.wait(); o4.wait()


def _ab_kernel(si_pad, ri_pad, c2f, s2f, c2b, s2b, dis_s, dis_r):
    ot = jax.ShapeDtypeStruct((E,), jnp.float32)
    return pl.kernel(
        _ab_body,
        out_type=(ot, ot, ot, ot),
        mesh=_mesh,
        compiler_params=_sc_params_nl,
        scratch_types=(
            [pltpu.VMEM((N,), jnp.float32)] * 2
            + [pltpu.VMEM((AB_PAD,), jnp.int32)] * 2
            + [pltpu.VMEM((AB_PAD,), jnp.float32)] * 4
            + [pltpu.VMEM((AB_PAD,), jnp.float32)] * 4
            + [pltpu.SemaphoreType.DMA]
        ),
    )(si_pad, ri_pad, c2f, s2f, c2b, s2b, dis_s, dis_r)


AB = 100
ABATCH = (E // NS) // AB
ABH = ABATCH // 2


def _scale_rows(xs, w_v, k):
    @pl.loop(0, 96, step=16)
    def _grp(g):
        wv = w_v[k, pl.ds(g, 16)]
        for j in range(16):
            wi = wv[j]
            for c in range(D // 16):
                sl = (g + j, pl.ds(c * 16, 16))
                xs[sl] = xs[sl] * wi

    wv = w_v[k, pl.ds(AB - 16, 16)]
    for j in range(12, 16):
        wi = wv[j]
        for c in range(D // 16):
            sl = (AB - 16 + j, pl.ds(c * 16, 16))
            xs[sl] = xs[sl] * wi


def _agg_phase(x_hbm, acc, gi_hbm, sx_hbm, w_hbm, row0,
               gi_v, sx_v, w_v, xs0, xs1,
               sem_i, sem_g0, sem_g1, sem_s0, sem_s1):
    for h in range(2):
        r = row0 + h * ABH
        c1 = pltpu.async_copy(gi_hbm.at[pl.ds(r, ABH)], gi_v, sem_i)
        c2 = pltpu.async_copy(sx_hbm.at[pl.ds(r, ABH)], sx_v, sem_i)
        c3 = pltpu.async_copy(w_hbm.at[pl.ds(r, ABH)], w_v, sem_i)
        c1.wait(); c2.wait(); c3.wait()

        pltpu.async_copy(x_hbm.at[gi_v.at[0]], xs0, sem_g0)
        pltpu.async_copy(x_hbm.at[gi_v.at[1]], xs1, sem_g1)

        @pl.loop(0, ABH // 2)
        def _it(t):
            k0 = 2 * t
            k1 = 2 * t + 1
            pltpu.make_async_copy(x_hbm.at[gi_v.at[k0]], xs0, sem_g0).wait()
            _scale_rows(xs0, w_v, k0)
            pltpu.async_copy(xs0, acc.at[sx_v.at[k0]], sem_s0, add=True)

            pltpu.make_async_copy(x_hbm.at[gi_v.at[k1]], xs1, sem_g1).wait()
            _scale_rows(xs1, w_v, k1)
            pltpu.async_copy(xs1, acc.at[sx_v.at[k1]], sem_s1, add=True)

            pltpu.make_async_copy(xs0, acc.at[sx_v.at[k0]], sem_s0).wait()

            @pl.when(t < ABH // 2 - 1)
            def _g0():
                pltpu.async_copy(x_hbm.at[gi_v.at[k0 + 2]], xs0, sem_g0)

            pltpu.make_async_copy(xs1, acc.at[sx_v.at[k1]], sem_s1).wait()

            @pl.when(t < ABH // 2 - 1)
            def _g1():
                pltpu.async_copy(x_hbm.at[gi_v.at[k1 + 2]], xs1, sem_g1)


def _agg_body(x_hbm, si_hbm, ri_hbm, af_hbm, bf_hbm, ab_hbm, bb_hbm, z_hbm,
              out_hbm,
              acc, si_v, ri_v, w_v, xs0, xs1,
              sem_i, sem_g0, sem_g1, sem_s0, sem_s1):
    cid = lax.axis_index("c")
    tid = lax.axis_index("s")

    r0 = tid * ROWS_PER_TILE
    pltpu.sync_copy(z_hbm.at[pl.ds(r0, ROWS_PER_TILE)],
                    acc.at[pl.ds(r0, ROWS_PER_TILE)])
    plsc.subcore_barrier()

    row0 = tid * ABATCH

    @pl.when(cid == 0)
    def _sc0():
        _agg_phase(x_hbm, acc, si_hbm, ri_hbm, af_hbm, row0,
                   si_v, ri_v, w_v, xs0, xs1,
                   sem_i, sem_g0, sem_g1, sem_s0, sem_s1)
        _agg_phase(x_hbm, acc, ri_hbm, si_hbm, ab_hbm, row0,
                   si_v, ri_v, w_v, xs0, xs1,
                   sem_i, sem_g0, sem_g1, sem_s0, sem_s1)

    @pl.when(cid == 1)
    def _sc1():
        _agg_phase(x_hbm, acc, si_hbm, ri_hbm, bf_hbm, row0,
                   si_v, ri_v, w_v, xs0, xs1,
                   sem_i, sem_g0, sem_g1, sem_s0, sem_s1)
        _agg_phase(x_hbm, acc, ri_hbm, si_hbm, bb_hbm, row0,
                   si_v, ri_v, w_v, xs0, xs1,
                   sem_i, sem_g0, sem_g1, sem_s0, sem_s1)

    plsc.subcore_barrier()
    pltpu.sync_copy(acc.at[pl.ds(r0, ROWS_PER_TILE)],
                    out_hbm.at[cid, pl.ds(r0, ROWS_PER_TILE)])


def _agg_kernel(x, si2d, ri2d, af2d, bf2d, ab2d, bb2d, z128):
    return pl.kernel(
        _agg_body,
        out_type=jax.ShapeDtypeStruct((NC, N, D), jnp.float32),
        mesh=_mesh,
        compiler_params=_sc_params,
        scratch_types=[
            pltpu.VMEM_SHARED((N, D), jnp.float32),
            pltpu.VMEM((ABH, AB), jnp.int32),
            pltpu.VMEM((ABH, AB), jnp.int32),
            pltpu.VMEM((ABH, AB), jnp.float32),
            pltpu.VMEM((AB, D), jnp.float32),
            pltpu.VMEM((AB, D), jnp.float32),
            pltpu.SemaphoreType.DMA,
            pltpu.SemaphoreType.DMA,
            pltpu.SemaphoreType.DMA,
            pltpu.SemaphoreType.DMA,
            pltpu.SemaphoreType.DMA,
        ],
    )(x, si2d, ri2d, af2d, bf2d, ab2d, bb2d, z128)


BN = 1000


def _dense1_body(s2d_ref, d2s_ref, x_ref, dd_ref, ws_ref, wd_ref,
                 bs_ref, bd_ref, o_ref):
    y = x_ref[...] * dd_ref[...]
    a = s2d_ref[...] + y
    b = d2s_ref[...] + y
    h = 0.5 * (lax.dot_general(a, ws_ref[...], (((1,), (0,)), ((), ())),
                               precision=lax.Precision.HIGHEST,
                               preferred_element_type=jnp.float32)
               + bs_ref[...])
    h = h + 0.5 * (lax.dot_general(b, wd_ref[...], (((1,), (0,)), ((), ())),
                                   precision=lax.Precision.HIGHEST,
                                   preferred_element_type=jnp.float32)
                   + bd_ref[...])
    o_ref[...] = jnp.maximum(h, 0.0)


def _dense2_body(s2d_ref, d2s_ref, x_ref, dd_ref, ws_ref, wd_ref,
                 bs_ref, bd_ref, wr_ref, br_ref, o_ref):
    y = x_ref[...] * dd_ref[...]
    a = s2d_ref[...] + y
    b = d2s_ref[...] + y
    h = 0.5 * (lax.dot_general(a, ws_ref[...], (((1,), (0,)), ((), ())),
                               precision=lax.Precision.HIGHEST,
                               preferred_element_type=jnp.float32)
               + bs_ref[...])
    h = h + 0.5 * (lax.dot_general(b, wd_ref[...], (((1,), (0,)), ((), ())),
                                   precision=lax.Precision.HIGHEST,
                                   preferred_element_type=jnp.float32)
                   + bd_ref[...])
    h = jnp.maximum(h, 0.0)
    o_ref[...] = lax.dot_general(h, wr_ref[...], (((1,), (0,)), ((), ())),
                                 precision=lax.Precision.HIGHEST,
                                 preferred_element_type=jnp.float32) + br_ref[...]


def _row_spec():
    return pl.BlockSpec((BN, D), lambda i: (i, 0))


def _w_spec():
    return pl.BlockSpec((D, D), lambda i: (0, 0))


def _b_spec():
    return pl.BlockSpec((1, D), lambda i: (0, 0))


def _dense1(s2d, d2s, x, dd2, ws, wd, bs, bd):
    return pl.pallas_call(
        _dense1_body,
        grid=(N // BN,),
        in_specs=[_row_spec(), _row_spec(), _row_spec(),
                  pl.BlockSpec((BN, 1), lambda i: (i, 0)),
                  _w_spec(), _w_spec(), _b_spec(), _b_spec()],
        out_specs=_row_spec(),
        out_shape=jax.ShapeDtypeStruct((N, D), jnp.float32),
    )(s2d, d2s, x, dd2, ws, wd, bs.reshape(1, D), bd.reshape(1, D))


def _dense2(s2d, d2s, x, dd2, ws, wd, bs, bd, wr, br):
    return pl.pallas_call(
        _dense2_body,
        grid=(N // BN,),
        in_specs=[_row_spec(), _row_spec(), _row_spec(),
                  pl.BlockSpec((BN, 1), lambda i: (i, 0)),
                  _w_spec(), _w_spec(), _b_spec(), _b_spec(),
                  _w_spec(), _b_spec()],
        out_specs=_row_spec(),
        out_shape=jax.ShapeDtypeStruct((N, D), jnp.float32),
    )(s2d, d2s, x, dd2, ws, wd, bs.reshape(1, D), bd.reshape(1, D),
      wr, br.reshape(1, D))


def kernel(x, edge_index, theta, W_s2d_0, W_d2s_0, b_s2d_0, b_d2s_0,
           W_s2d_1, W_d2s_1, b_s2d_1, b_d2s_1, W_read, b_read):
    si = edge_index[0].astype(jnp.int32)
    ri = edge_index[1].astype(jnp.int32)

    c2f = jnp.cos(theta) ** 2
    s2f = jnp.sin(theta) ** 2
    c2b = jnp.cos(jnp.pi / 2 - theta) ** 2
    s2b = jnp.sin(jnp.pi / 2 - theta) ** 2

    zpad = jnp.zeros((E, 14), jnp.float32)
    rows_s = jnp.concatenate([c2f[:, None], s2f[:, None], zpad], axis=1)
    rows_r = jnp.concatenate([c2b[:, None], s2b[:, None], zpad], axis=1)

    si2d = si.reshape(E // DB, DB)
    ri2d = ri.reshape(E // DB, DB)
    z16 = jnp.zeros((N, 16), jnp.float32)

    degp = _deg_kernel(si2d, ri2d, rows_s, rows_r, z16)
    degs = degp[0] + degp[1]
    deg_s = degs[:, 0] + 1.0 + 1e-12
    deg_r = degs[:, 1] + 1.0 + 1e-12
    dis_s = jnp.where(deg_s > 1e11, 0.0, lax.rsqrt(deg_s))
    dis_r = jnp.where(deg_r > 1e11, 0.0, lax.rsqrt(deg_r))
    dd = dis_s * dis_r

    ipad = jnp.zeros((16,), jnp.int32)
    fpad = jnp.zeros((16,), jnp.float32)
    si_pad = jnp.concatenate([si, ipad])
    ri_pad = jnp.concatenate([ri, ipad])
    c2f_p = jnp.concatenate([c2f, fpad])
    s2f_p = jnp.concatenate([s2f, fpad])
    c2b_p = jnp.concatenate([c2b, fpad])
    s2b_p = jnp.concatenate([s2b, fpad])

    af, bf, ab_, bb = _ab_kernel(si_pad, ri_pad, c2f_p, s2f_p, c2b_p, s2b_p,
                                 dis_s, dis_r)

    z128 = jnp.zeros((N, D), jnp.float32)
    dd2 = dd.reshape(N, 1)

    si2a = si.reshape(E // AB, AB)
    ri2a = ri.reshape(E // AB, AB)
    af2 = af.reshape(E // AB, AB)
    bf2 = bf.reshape(E // AB, AB)
    ab2 = ab_.reshape(E // AB, AB)
    bb2 = bb.reshape(E // AB, AB)

    aggs = _agg_kernel(x, si2a, ri2a, af2, bf2, ab2, bb2, z128)
    x1 = _dense1(aggs[0], aggs[1], x, dd2, W_s2d_0, W_d2s_0, b_s2d_0, b_d2s_0)

    aggs2 = _agg_kernel(x1, si2a, ri2a, af2, bf2, ab2, bb2, z128)
    out = _dense2(aggs2[0], aggs2[1], x1, dd2, W_s2d_1, W_d2s_1,
                  b_s2d_1, b_d2s_1, W_read, b_read)
    return out


# speedup vs baseline: 2.5421x; 1.0121x over previous
import jax
import jax.numpy as jnp
from jax import lax
from jax.experimental import pallas as pl
from jax.experimental.pallas import tpu as pltpu
from jax.experimental.pallas import tpu_sc as plsc

N = 10000
E = 160000
D = 128
NC = 2
NS = 16
NW = NC * NS

ROWS_PER_TILE = N // NS

AB = 100
ABATCH = (E // NS) // AB
ABH = ABATCH // 2

DEG_E_PER_TILE = E // NW
DB = 100
DSUP = 2500
DNSUP = DEG_E_PER_TILE // DSUP
DSCAT = DSUP // DB

AB_E_PER_TILE = E // NW
AB_PAD = 5008

_mesh = plsc.VectorSubcoreMesh(core_axis_name="c", subcore_axis_name="s")
_sc_params = pltpu.CompilerParams(use_tc_tiling_on_sc=False)
_sc_params_nl = pltpu.CompilerParams(use_tc_tiling_on_sc=False, needs_layout_passes=False)


def _wid():
    return lax.axis_index("c") * NS + lax.axis_index("s")


def _deg_body(si_hbm, ri_hbm, rows_s_hbm, rows_r_hbm, z16_hbm, out_hbm,
              acc, idx_s_v, idx_r_v, rs_v, rr_v, sem_i, sem_r, sem_sc):
    cid = lax.axis_index("c")
    tid = lax.axis_index("s")
    wid = cid * NS + tid

    r0 = tid * ROWS_PER_TILE
    pltpu.sync_copy(z16_hbm.at[pl.ds(r0, ROWS_PER_TILE)],
                    acc.at[pl.ds(r0, ROWS_PER_TILE)])
    plsc.subcore_barrier()

    base = wid * DEG_E_PER_TILE

    @pl.loop(0, DNSUP)
    def _sup(sup):
        e0 = base + sup * DSUP
        row0 = e0 // DB
        c1 = pltpu.async_copy(si_hbm.at[pl.ds(row0, DSCAT)], idx_s_v, sem_i)
        c2 = pltpu.async_copy(ri_hbm.at[pl.ds(row0, DSCAT)], idx_r_v, sem_i)
        c3 = pltpu.async_copy(rows_s_hbm.at[pl.ds(e0, DSUP)], rs_v, sem_r)
        c4 = pltpu.async_copy(rows_r_hbm.at[pl.ds(e0, DSUP)], rr_v, sem_r)
        c1.wait(); c2.wait(); c3.wait(); c4.wait()

        @pl.loop(0, DSCAT)
        def _sc(j):
            pltpu.async_copy(rs_v.at[pl.ds(j * DB, DB)],
                             acc.at[idx_s_v.at[j]], sem_sc, add=True)
            pltpu.async_copy(rr_v.at[pl.ds(j * DB, DB)],
                             acc.at[idx_r_v.at[j]], sem_sc, add=True)

        @pl.loop(0, DSCAT)
        def _dr(j):
            pltpu.make_async_copy(rs_v.at[pl.ds(j * DB, DB)],
                                  acc.at[idx_s_v.at[j]], sem_sc).wait()
            pltpu.make_async_copy(rr_v.at[pl.ds(j * DB, DB)],
                                  acc.at[idx_r_v.at[j]], sem_sc).wait()

    plsc.subcore_barrier()
    pltpu.sync_copy(acc.at[pl.ds(r0, ROWS_PER_TILE)],
                    out_hbm.at[cid, pl.ds(r0, ROWS_PER_TILE)])


def _deg_kernel(si2d, ri2d, rows_s, rows_r, z16):
    return pl.kernel(
        _deg_body,
        out_type=jax.ShapeDtypeStruct((NC, N, 16), jnp.float32),
        mesh=_mesh,
        compiler_params=_sc_params,
        scratch_types=[
            pltpu.VMEM_SHARED((N, 16), jnp.float32),
            pltpu.VMEM((DSCAT, DB), jnp.int32),
            pltpu.VMEM((DSCAT, DB), jnp.int32),
            pltpu.VMEM((DSUP, 16), jnp.float32),
            pltpu.VMEM((DSUP, 16), jnp.float32),
            pltpu.SemaphoreType.DMA,
            pltpu.SemaphoreType.DMA,
            pltpu.SemaphoreType.DMA,
        ],
    )(si2d, ri2d, rows_s, rows_r, z16)


def _ab_body(si_hbm, ri_hbm, c2f_hbm, s2f_hbm,
             dis_s_hbm, dis_r_hbm,
             af_hbm, bf_hbm, ab_hbm, bb_hbm,
             ds_v, dr_v, si_v, ri_v, tf_v, tg_v,
             af_v, bf_v, ab_v, bb_v, sem):
    wid = _wid()
    base = wid * AB_E_PER_TILE

    pltpu.async_copy(dis_s_hbm, ds_v, sem).wait()
    pltpu.async_copy(dis_r_hbm, dr_v, sem).wait()
    c1 = pltpu.async_copy(si_hbm.at[pl.ds(base, AB_PAD)], si_v, sem)
    c2 = pltpu.async_copy(ri_hbm.at[pl.ds(base, AB_PAD)], ri_v, sem)
    c3 = pltpu.async_copy(c2f_hbm.at[pl.ds(base, AB_PAD)], tf_v, sem)
    c4 = pltpu.async_copy(s2f_hbm.at[pl.ds(base, AB_PAD)], tg_v, sem)
    c1.wait(); c2.wait(); c3.wait(); c4.wait()

    @pl.loop(0, AB_PAD // 16)
    def _ck(c):
        sl = pl.ds(c * 16, 16)
        sv = si_v[sl]
        rv = ri_v[sl]
        dss = plsc.load_gather(ds_v, [sv])
        dsr = plsc.load_gather(ds_v, [rv])
        drs = plsc.load_gather(dr_v, [sv])
        drr = plsc.load_gather(dr_v, [rv])
        af_v[sl] = dss * tf_v[sl] * drr
        bf_v[sl] = drs * tg_v[sl] * dsr
        ab_v[sl] = dsr * tg_v[sl] * drs
        bb_v[sl] = drr * tf_v[sl] * dss

    o1 = pltpu.async_copy(af_v.at[pl.ds(0, AB_E_PER_TILE)],
                          af_hbm.at[pl.ds(base, AB_E_PER_TILE)], sem)
    o2 = pltpu.async_copy(bf_v.at[pl.ds(0, AB_E_PER_TILE)],
                          bf_hbm.at[pl.ds(base, AB_E_PER_TILE)], sem)
    o3 = pltpu.async_copy(ab_v.at[pl.ds(0, AB_E_PER_TILE)],
                          ab_hbm.at[pl.ds(base, AB_E_PER_TILE)], sem)
    o4 = pltpu.async_copy(bb_v.at[pl.ds(0, AB_E_PER_TILE)],
                          bb_hbm.at[pl.ds(base, AB_E_PER_TILE)], sem)
    o1.wait(); o2.wait(); o3.wait(); o4.wait()


def _ab_kernel(si_pad, ri_pad, c2f, s2f, dis_s, dis_r):
    ot = jax.ShapeDtypeStruct((E,), jnp.float32)
    return pl.kernel(
        _ab_body,
        out_type=(ot, ot, ot, ot),
        mesh=_mesh,
        compiler_params=_sc_params_nl,
        scratch_types=(
            [pltpu.VMEM((N,), jnp.float32)] * 2
            + [pltpu.VMEM((AB_PAD,), jnp.int32)] * 2
            + [pltpu.VMEM((AB_PAD,), jnp.float32)] * 2
            + [pltpu.VMEM((AB_PAD,), jnp.float32)] * 4
            + [pltpu.SemaphoreType.DMA]
        ),
    )(si_pad, ri_pad, c2f, s2f, dis_s, dis_r)


AB = 100
ABATCH = (E // NS) // AB
ABH = ABATCH // 2


def _scale_rows(xs, w_v, k):
    @pl.loop(0, 96, step=16)
    def _grp(g):
        wv = w_v[k, pl.ds(g, 16)]
        for j in range(16):
            wi = wv[j]
            for c in range(D // 16):
                sl = (g + j, pl.ds(c * 16, 16))
                xs[sl] = xs[sl] * wi

    wv = w_v[k, pl.ds(AB - 16, 16)]
    for j in range(12, 16):
        wi = wv[j]
        for c in range(D // 16):
            sl = (AB - 16 + j, pl.ds(c * 16, 16))
            xs[sl] = xs[sl] * wi


def _agg_phase(x_hbm, acc, gi_hbm, sx_hbm, w_hbm, row0,
               gi_v, sx_v, w_v, xs0, xs1,
               sem_i, sem_g0, sem_g1, sem_s0, sem_s1):
    for h in range(2):
        r = row0 + h * ABH
        c1 = pltpu.async_copy(gi_hbm.at[pl.ds(r, ABH)], gi_v, sem_i)
        c2 = pltpu.async_copy(sx_hbm.at[pl.ds(r, ABH)], sx_v, sem_i)
        c3 = pltpu.async_copy(w_hbm.at[pl.ds(r, ABH)], w_v, sem_i)
        c1.wait(); c2.wait(); c3.wait()

        pltpu.async_copy(x_hbm.at[gi_v.at[0]], xs0, sem_g0)
        pltpu.async_copy(x_hbm.at[gi_v.at[1]], xs1, sem_g1)

        @pl.loop(0, ABH // 2)
        def _it(t):
            k0 = 2 * t
            k1 = 2 * t + 1
            pltpu.make_async_copy(x_hbm.at[gi_v.at[k0]], xs0, sem_g0).wait()
            _scale_rows(xs0, w_v, k0)
            pltpu.async_copy(xs0, acc.at[sx_v.at[k0]], sem_s0, add=True)

            pltpu.make_async_copy(x_hbm.at[gi_v.at[k1]], xs1, sem_g1).wait()
            _scale_rows(xs1, w_v, k1)
            pltpu.async_copy(xs1, acc.at[sx_v.at[k1]], sem_s1, add=True)

            pltpu.make_async_copy(xs0, acc.at[sx_v.at[k0]], sem_s0).wait()

            @pl.when(t < ABH // 2 - 1)
            def _g0():
                pltpu.async_copy(x_hbm.at[gi_v.at[k0 + 2]], xs0, sem_g0)

            pltpu.make_async_copy(xs1, acc.at[sx_v.at[k1]], sem_s1).wait()

            @pl.when(t < ABH // 2 - 1)
            def _g1():
                pltpu.async_copy(x_hbm.at[gi_v.at[k1 + 2]], xs1, sem_g1)


def _agg_body(x_hbm, si_hbm, ri_hbm, af_hbm, bf_hbm, ab_hbm, bb_hbm, z_hbm,
              out_hbm,
              acc, si_v, ri_v, w_v, xs0, xs1,
              sem_i, sem_g0, sem_g1, sem_s0, sem_s1):
    cid = lax.axis_index("c")
    tid = lax.axis_index("s")

    r0 = tid * ROWS_PER_TILE
    pltpu.sync_copy(z_hbm.at[pl.ds(r0, ROWS_PER_TILE)],
                    acc.at[pl.ds(r0, ROWS_PER_TILE)])
    plsc.subcore_barrier()

    row0 = tid * ABATCH

    @pl.when(cid == 0)
    def _sc0():
        _agg_phase(x_hbm, acc, si_hbm, ri_hbm, af_hbm, row0,
                   si_v, ri_v, w_v, xs0, xs1,
                   sem_i, sem_g0, sem_g1, sem_s0, sem_s1)
        _agg_phase(x_hbm, acc, ri_hbm, si_hbm, ab_hbm, row0,
                   si_v, ri_v, w_v, xs0, xs1,
                   sem_i, sem_g0, sem_g1, sem_s0, sem_s1)

    @pl.when(cid == 1)
    def _sc1():
        _agg_phase(x_hbm, acc, si_hbm, ri_hbm, bf_hbm, row0,
                   si_v, ri_v, w_v, xs0, xs1,
                   sem_i, sem_g0, sem_g1, sem_s0, sem_s1)
        _agg_phase(x_hbm, acc, ri_hbm, si_hbm, bb_hbm, row0,
                   si_v, ri_v, w_v, xs0, xs1,
                   sem_i, sem_g0, sem_g1, sem_s0, sem_s1)

    plsc.subcore_barrier()
    pltpu.sync_copy(acc.at[pl.ds(r0, ROWS_PER_TILE)],
                    out_hbm.at[cid, pl.ds(r0, ROWS_PER_TILE)])


def _agg_kernel(x, si2d, ri2d, af2d, bf2d, ab2d, bb2d, z128):
    return pl.kernel(
        _agg_body,
        out_type=jax.ShapeDtypeStruct((NC, N, D), jnp.float32),
        mesh=_mesh,
        compiler_params=_sc_params,
        scratch_types=[
            pltpu.VMEM_SHARED((N, D), jnp.float32),
            pltpu.VMEM((ABH, AB), jnp.int32),
            pltpu.VMEM((ABH, AB), jnp.int32),
            pltpu.VMEM((ABH, AB), jnp.float32),
            pltpu.VMEM((AB, D), jnp.float32),
            pltpu.VMEM((AB, D), jnp.float32),
            pltpu.SemaphoreType.DMA,
            pltpu.SemaphoreType.DMA,
            pltpu.SemaphoreType.DMA,
            pltpu.SemaphoreType.DMA,
            pltpu.SemaphoreType.DMA,
        ],
    )(x, si2d, ri2d, af2d, bf2d, ab2d, bb2d, z128)


BN = 1000


def _dense1_body(s2d_ref, d2s_ref, x_ref, dd_ref, ws_ref, wd_ref,
                 bs_ref, bd_ref, o_ref):
    y = x_ref[...] * dd_ref[...]
    a = s2d_ref[...] + y
    b = d2s_ref[...] + y
    h = 0.5 * (lax.dot_general(a, ws_ref[...], (((1,), (0,)), ((), ())),
                               precision=lax.Precision.HIGHEST,
                               preferred_element_type=jnp.float32)
               + bs_ref[...])
    h = h + 0.5 * (lax.dot_general(b, wd_ref[...], (((1,), (0,)), ((), ())),
                                   precision=lax.Precision.HIGHEST,
                                   preferred_element_type=jnp.float32)
                   + bd_ref[...])
    o_ref[...] = jnp.maximum(h, 0.0)


def _dense2_body(s2d_ref, d2s_ref, x_ref, dd_ref, ws_ref, wd_ref,
                 bs_ref, bd_ref, wr_ref, br_ref, o_ref):
    y = x_ref[...] * dd_ref[...]
    a = s2d_ref[...] + y
    b = d2s_ref[...] + y
    h = 0.5 * (lax.dot_general(a, ws_ref[...], (((1,), (0,)), ((), ())),
                               precision=lax.Precision.HIGHEST,
                               preferred_element_type=jnp.float32)
               + bs_ref[...])
    h = h + 0.5 * (lax.dot_general(b, wd_ref[...], (((1,), (0,)), ((), ())),
                                   precision=lax.Precision.HIGHEST,
                                   preferred_element_type=jnp.float32)
                   + bd_ref[...])
    h = jnp.maximum(h, 0.0)
    o_ref[...] = lax.dot_general(h, wr_ref[...], (((1,), (0,)), ((), ())),
                                 precision=lax.Precision.HIGHEST,
                                 preferred_element_type=jnp.float32) + br_ref[...]


def _row_spec():
    return pl.BlockSpec((BN, D), lambda i: (i, 0))


def _w_spec():
    return pl.BlockSpec((D, D), lambda i: (0, 0))


def _b_spec():
    return pl.BlockSpec((1, D), lambda i: (0, 0))


def _dense1(s2d, d2s, x, dd2, ws, wd, bs, bd):
    return pl.pallas_call(
        _dense1_body,
        grid=(N // BN,),
        in_specs=[_row_spec(), _row_spec(), _row_spec(),
                  pl.BlockSpec((BN, 1), lambda i: (i, 0)),
                  _w_spec(), _w_spec(), _b_spec(), _b_spec()],
        out_specs=_row_spec(),
        out_shape=jax.ShapeDtypeStruct((N, D), jnp.float32),
    )(s2d, d2s, x, dd2, ws, wd, bs.reshape(1, D), bd.reshape(1, D))


def _dense2(s2d, d2s, x, dd2, ws, wd, bs, bd, wr, br):
    return pl.pallas_call(
        _dense2_body,
        grid=(N // BN,),
        in_specs=[_row_spec(), _row_spec(), _row_spec(),
                  pl.BlockSpec((BN, 1), lambda i: (i, 0)),
                  _w_spec(), _w_spec(), _b_spec(), _b_spec(),
                  _w_spec(), _b_spec()],
        out_specs=_row_spec(),
        out_shape=jax.ShapeDtypeStruct((N, D), jnp.float32),
    )(s2d, d2s, x, dd2, ws, wd, bs.reshape(1, D), bd.reshape(1, D),
      wr, br.reshape(1, D))


def kernel(x, edge_index, theta, W_s2d_0, W_d2s_0, b_s2d_0, b_d2s_0,
           W_s2d_1, W_d2s_1, b_s2d_1, b_d2s_1, W_read, b_read):
    si = edge_index[0].astype(jnp.int32)
    ri = edge_index[1].astype(jnp.int32)

    c2f = jnp.cos(theta) ** 2
    s2f = jnp.sin(theta) ** 2

    zpad = jnp.zeros((E, 14), jnp.float32)
    rows_s = jnp.concatenate([c2f[:, None], s2f[:, None], zpad], axis=1)
    rows_r = jnp.concatenate([s2f[:, None], c2f[:, None], zpad], axis=1)

    si2d = si.reshape(E // DB, DB)
    ri2d = ri.reshape(E // DB, DB)
    z16 = jnp.zeros((N, 16), jnp.float32)

    degp = _deg_kernel(si2d, ri2d, rows_s, rows_r, z16)
    degs = degp[0] + degp[1]
    deg_s = degs[:, 0] + 1.0 + 1e-12
    deg_r = degs[:, 1] + 1.0 + 1e-12
    dis_s = jnp.where(deg_s > 1e11, 0.0, lax.rsqrt(deg_s))
    dis_r = jnp.where(deg_r > 1e11, 0.0, lax.rsqrt(deg_r))
    dd = dis_s * dis_r

    ipad = jnp.zeros((16,), jnp.int32)
    fpad = jnp.zeros((16,), jnp.float32)
    si_pad = jnp.concatenate([si, ipad])
    ri_pad = jnp.concatenate([ri, ipad])
    c2f_p = jnp.concatenate([c2f, fpad])
    s2f_p = jnp.concatenate([s2f, fpad])

    af, bf, ab_, bb = _ab_kernel(si_pad, ri_pad, c2f_p, s2f_p,
                                 dis_s, dis_r)

    z128 = jnp.zeros((N, D), jnp.float32)
    dd2 = dd.reshape(N, 1)

    si2a = si.reshape(E // AB, AB)
    ri2a = ri.reshape(E // AB, AB)
    af2 = af.reshape(E // AB, AB)
    bf2 = bf.reshape(E // AB, AB)
    ab2 = ab_.reshape(E // AB, AB)
    bb2 = bb.reshape(E // AB, AB)

    aggs = _agg_kernel(x, si2a, ri2a, af2, bf2, ab2, bb2, z128)
    x1 = _dense1(aggs[0], aggs[1], x, dd2, W_s2d_0, W_d2s_0, b_s2d_0, b_d2s_0)

    aggs2 = _agg_kernel(x1, si2a, ri2a, af2, bf2, ab2, bb2, z128)
    out = _dense2(aggs2[0], aggs2[1], x1, dd2, W_s2d_1, W_d2s_1,
                  b_s2d_1, b_d2s_1, W_read, b_read)
    return out
